# baseline (speedup 1.0000x reference)
import functools
from typing import Any

import jax
import jax.numpy as jnp
from jax import lax
from jax.experimental import pallas as pl
from jax.experimental.pallas import tpu as pltpu
from jax.experimental.pallas import tpu_sc as plsc

N = 10000
G = 16
NC = 2
NS = 16
NW = NC * NS
CH = 128
RPT = 640
NP = NS * RPT

_f32 = jnp.float32
_mesh = plsc.VectorSubcoreMesh(core_axis_name="c", subcore_axis_name="s")
_sc_params = pltpu.CompilerParams(use_tc_tiling_on_sc=False)


def _fill_1d(ref, n, val):
    def body(i, _):
        ref[pl.ds(i * 16, 16)] = jnp.full((16,), val, _f32)
        return _
    lax.fori_loop(0, n // 16, body, 0)


def _fill_rows(ref, nrows, d, val):
    def body(r, _):
        for c in range(d // 16):
            ref[r, pl.ds(c * 16, 16)] = jnp.full((16,), val, _f32)
        return _
    lax.fori_loop(0, nrows, body, 0)


def _wid(cid, sid):
    return sid * NC + cid


def _deg_body(cpt, dstp, out, deg_sh, dstv, ones_v, zer_v, sem):
    cid = lax.axis_index("c")
    sid = lax.axis_index("s")
    w = _wid(cid, sid)
    _fill_1d(zer_v, RPT, 0.0)
    _fill_1d(ones_v, CH, 1.0)
    pltpu.sync_copy(zer_v, deg_sh.at[pl.ds(sid * RPT, RPT)])
    plsc.subcore_barrier()
    pltpu.sync_copy(dstp.at[pl.ds(w * cpt, cpt)], dstv)

    def step(j, _):
        pltpu.sync_copy(ones_v, deg_sh.at[dstv.at[j]], add=True)
        return _
    lax.fori_loop(0, cpt, step, 0)
    plsc.subcore_barrier()
    pltpu.sync_copy(deg_sh.at[pl.ds(sid * RPT, RPT)],
                    out.at[pl.ds(cid * NP + sid * RPT, RPT)])


def _deg_call(dstp, cpt):
    return pl.kernel(
        functools.partial(_deg_body, cpt),
        out_type=jax.ShapeDtypeStruct((NC * NP,), _f32),
        mesh=_mesh,
        compiler_params=_sc_params,
        scratch_types=[
            pltpu.VMEM_SHARED((NP,), _f32),
            pltpu.VMEM((cpt, CH), jnp.int32),
            pltpu.VMEM((CH,), _f32),
            pltpu.VMEM((RPT,), _f32),
            pltpu.SemaphoreType.DMA,
        ],
    )(dstp)


_SS = 8


def _gcn_body(cpt, d, g, srcp, dstp, out, acc_sh, srcw, dstw, rows0, rows1,
              semi, sem0, sem1):
    cid = lax.axis_index("c")
    sid = lax.axis_index("s")
    w = _wid(cid, sid)
    nss = cpt // _SS
    _fill_rows(rows0, CH, d, 0.0)
    for r in range(RPT // CH):
        pltpu.sync_copy(rows0, acc_sh.at[pl.ds(sid * RPT + r * CH, CH)])
    plsc.subcore_barrier()
    pltpu.sync_copy(srcp.at[pl.ds(w * cpt, _SS)], srcw.at[0])
    pltpu.sync_copy(dstp.at[pl.ds(w * cpt, _SS)], dstw.at[0])

    def ss_body(p, _):
        b = p % 2
        nb = 1 - b
        off = w * cpt + jnp.minimum((p + 1) * _SS, cpt - _SS)
        pltpu.async_copy(srcp.at[pl.ds(off, _SS)], srcw.at[nb], semi)
        pltpu.async_copy(dstp.at[pl.ds(off, _SS)], dstw.at[nb], semi)
        bufs = (rows0, rows1)
        sems = (sem0, sem1)
        pltpu.async_copy(g.at[srcw.at[b, 0]], rows0, sem0)
        pltpu.async_copy(g.at[srcw.at[b, 1]], rows1, sem1)
        for k in range(_SS):
            rb, smb = bufs[k % 2], sems[k % 2]
            pltpu.make_async_copy(g.at[srcw.at[b, k]], rb, smb).wait()
            pltpu.sync_copy(rb, acc_sh.at[dstw.at[b, k]], add=True)
            if k + 2 < _SS:
                pltpu.async_copy(g.at[srcw.at[b, k + 2]], rb, smb)
        pltpu.make_async_copy(srcp.at[pl.ds(off, _SS)], srcw.at[nb],
                              semi).wait()
        pltpu.make_async_copy(dstp.at[pl.ds(off, _SS)], dstw.at[nb],
                              semi).wait()
        return _
    lax.fori_loop(0, nss, ss_body, 0)
    plsc.subcore_barrier()
    for r in range(RPT // CH):
        pltpu.sync_copy(acc_sh.at[pl.ds(sid * RPT + r * CH, CH)],
                        out.at[pl.ds(cid * NP + sid * RPT + r * CH, CH)])


def _gcn_call(g, srcp, dstp, cpt, d):
    return pl.kernel(
        functools.partial(_gcn_body, cpt, d),
        out_type=jax.ShapeDtypeStruct((NC * NP, d), _f32),
        mesh=_mesh,
        compiler_params=_sc_params,
        scratch_types=[
            pltpu.VMEM_SHARED((NP, d), _f32),
            pltpu.VMEM((2, _SS, CH), jnp.int32),
            pltpu.VMEM((2, _SS, CH), jnp.int32),
            pltpu.VMEM((CH, d), _f32),
            pltpu.VMEM((CH, d), _f32),
            pltpu.SemaphoreType.DMA,
            pltpu.SemaphoreType.DMA,
            pltpu.SemaphoreType.DMA,
        ],
    )(g, srcp, dstp)


_NB = 4


def _gat_body(cpt, d, asp, adp, qh, t, srcp, dstp, den_out, acc_out,
              den_sh, acc_sh, srcv, dstv,
              asv0, asv1, asv2, asv3, adv0, adv1, adv2, adv3,
              exv0, exv1, exv2, exv3, rows0, rows1, rows2, rows3,
              qv, zer_v, semg0, semg1, semg2, semg3, semd, semr):
    cid = lax.axis_index("c")
    sid = lax.axis_index("s")
    w = _wid(cid, sid)
    asv = (asv0, asv1, asv2, asv3)
    adv = (adv0, adv1, adv2, adv3)
    exv = (exv0, exv1, exv2, exv3)
    rows = (rows0, rows1, rows2, rows3)
    semg = (semg0, semg1, semg2, semg3)
    _fill_1d(zer_v, RPT, 0.0)
    pltpu.sync_copy(zer_v, den_sh.at[pl.ds(sid * RPT, RPT)])
    _fill_rows(rows0, CH, d, 0.0)
    for r in range(RPT // CH):
        pltpu.sync_copy(rows0, acc_sh.at[pl.ds(sid * RPT + r * CH, CH)])
    plsc.subcore_barrier()
    pltpu.sync_copy(srcp.at[pl.ds(w * cpt, cpt)], srcv.at[pl.ds(0, cpt)])
    pltpu.sync_copy(dstp.at[pl.ds(w * cpt, cpt)], dstv.at[pl.ds(0, cpt)])
    for rr in range(cpt, cpt + _NB):
        for c in range(CH // 16):
            srcv[rr, pl.ds(c * 16, 16)] = jnp.zeros((16,), jnp.int32)
            dstv[rr, pl.ds(c * 16, 16)] = jnp.zeros((16,), jnp.int32)
    pltpu.sync_copy(qh, qv)
    qq = qv[...]

    def start(j, k):
        pltpu.async_copy(asp.at[srcv.at[j]], asv[k], semg[k])
        pltpu.async_copy(adp.at[dstv.at[j]], adv[k], semg[k])
        pltpu.async_copy(t.at[srcv.at[j]], rows[k], semg[k])

    def wait_g(j, k):
        pltpu.make_async_copy(asp.at[srcv.at[j]], asv[k], semg[k]).wait()
        pltpu.make_async_copy(adp.at[dstv.at[j]], adv[k], semg[k]).wait()
        pltpu.make_async_copy(t.at[srcv.at[j]], rows[k], semg[k]).wait()

    for k in range(_NB):
        start(k, k)

    def step(p, _):
        j0 = p * _NB
        for k in range(_NB):
            j = j0 + k
            wait_g(j, k)
            for gidx in range(CH // 16):
                sl = pl.ds(gidx * 16, 16)
                u = asv[k][sl] + adv[k][sl]
                e = jnp.where(u > 0, u, 0.2 * u)
                ad = adv[k][sl]
                c_sh = jnp.where(ad > 0, ad, 0.2 * ad) + qq
                exv[k][sl] = jnp.exp(e - c_sh)
            pltpu.async_copy(exv[k], den_sh.at[dstv.at[j]], semd, add=True)

            def scale(gi, _c, k=k):
                exg = exv[k][pl.ds(gi * 16, 16)]
                for l in range(16):
                    kk = gi * 16 + l
                    exb = jnp.full((16,), 1.0, _f32) * exg[l]
                    for c in range(d // 16):
                        csl = pl.ds(c * 16, 16)
                        rows[k][kk, csl] = rows[k][kk, csl] * exb
                return _c
            lax.fori_loop(0, CH // 16, scale, 0)
            pltpu.async_copy(rows[k], acc_sh.at[dstv.at[j]], semr, add=True)
        for k in range(_NB):
            j = j0 + k
            pltpu.make_async_copy(exv[k], den_sh.at[dstv.at[j]],
                                  semd).wait()
            pltpu.make_async_copy(rows[k], acc_sh.at[dstv.at[j]],
                                  semr).wait()
        for k in range(_NB):
            start(j0 + _NB + k, k)
        return _
    lax.fori_loop(0, cpt // _NB, step, 0)
    for k in range(_NB):
        wait_g(cpt + k, k)
    plsc.subcore_barrier()
    pltpu.sync_copy(den_sh.at[pl.ds(sid * RPT, RPT)],
                    den_out.at[pl.ds(cid * NP + sid * RPT, RPT)])
    for r in range(RPT // CH):
        pltpu.sync_copy(acc_sh.at[pl.ds(sid * RPT + r * CH, CH)],
                        acc_out.at[pl.ds(cid * NP + sid * RPT + r * CH, CH)])


def _gat_call(asp, adp, qh, t, srcp, dstp, cpt, d):
    return pl.kernel(
        functools.partial(_gat_body, cpt, d),
        out_type=(jax.ShapeDtypeStruct((NC * NP,), _f32),
                  jax.ShapeDtypeStruct((NC * NP, d), _f32)),
        mesh=_mesh,
        compiler_params=_sc_params,
        scratch_types=[
            pltpu.VMEM_SHARED((NP,), _f32),
            pltpu.VMEM_SHARED((NP, d), _f32),
            pltpu.VMEM((cpt + _NB, CH), jnp.int32),
            pltpu.VMEM((cpt + _NB, CH), jnp.int32),
        ] + [pltpu.VMEM((CH,), _f32)] * 12 + [
            pltpu.VMEM((CH, d), _f32),
            pltpu.VMEM((CH, d), _f32),
            pltpu.VMEM((CH, d), _f32),
            pltpu.VMEM((CH, d), _f32),
            pltpu.VMEM((16,), _f32),
            pltpu.VMEM((RPT,), _f32),
        ] + [pltpu.SemaphoreType.DMA] * 6,
    )(asp, adp, qh, t, srcp, dstp)


def _leaky01(v):
    return jnp.where(v > 0, v, 0.01 * v)


def _leaky20(v):
    return jnp.where(v > 0, v, 0.2 * v)


def _k2_body(x_ref, w1_ref, dega_ref, degb_ref, g1_ref, dis_ref):
    deg = dega_ref[...] + degb_ref[...] + 1.0
    dis = lax.rsqrt(deg)
    p1 = jnp.dot(x_ref[...], w1_ref[...], preferred_element_type=_f32)
    g1_ref[...] = dis * p1
    dis_ref[...] = dis


def _k4_body(acc_a, acc_b, g1, dis, b1, w2, a_s, a_d, t_o, as_o, ad_o,
             adq_o, q_o):
    h1 = _leaky01(dis[...] * (acc_a[...] + acc_b[...] + g1[...])
                  + b1[...][None, :])
    t = jnp.dot(h1, w2[...], preferred_element_type=_f32)
    asv = jnp.dot(t, a_s[...][:, None], preferred_element_type=_f32)
    adv = jnp.dot(t, a_d[...][:, None], preferred_element_type=_f32)
    m = jnp.max(asv)
    q = jnp.maximum(m, 0.2 * m)
    t_o[...] = t
    as_o[...] = asv
    ad_o[...] = adv
    adq_o[...] = _leaky20(adv) + q
    q_o[...] = jnp.full((16,), 1.0, _f32) * q


def _k6_body(acc_a, acc_b, den_a, den_b, t, asv, adv, adqv, b2, w3, dis,
             g3_o):
    exs = jnp.exp(_leaky20(asv[...] + adv[...]) - adqv[...])
    den = jnp.maximum(den_a[...] + den_b[...] + exs, 1e-16)
    num = acc_a[...] + acc_b[...] + exs * t[...]
    h2 = _leaky01(num / den + b2[...][None, :])
    p3 = jnp.dot(h2, w3[...], preferred_element_type=_f32)
    g3_o[...] = dis[...] * p3


def _k8_body(acc_a, acc_b, g3, dis, b3, batch, wl, bl, out):
    h3 = _leaky01(dis[...] * (acc_a[...] + acc_b[...] + g3[...])
                  + b3[...][None, :])
    oh = (batch[...] == lax.broadcasted_iota(jnp.int32, (N, G), 1))
    oh = oh.astype(_f32)
    sums = lax.dot_general(oh, h3, (((0,), (0,)), ((), ())),
                           preferred_element_type=_f32)
    cnt = jnp.sum(oh, axis=0)[:, None]
    pooled = sums / jnp.maximum(cnt, 1.0)
    out[...] = jnp.dot(pooled, wl[...], preferred_element_type=_f32) \
        + bl[...][None, :]


def _tc(body, out_shape, *args):
    return pl.pallas_call(body, out_shape=out_shape)(*args)


def kernel(x, edge_index, batch, W1, b1, W2, a_src, a_dst, b2, W3, b3, Wl,
           bl):
    e = edge_index.shape[1]
    cpt = -(-e // (CH * NW))
    cpt = -(-cpt // 8) * 8
    epad = cpt * NW * CH
    pad = epad - e
    ar = jnp.arange(pad, dtype=jnp.int32)
    srcp = jnp.concatenate([edge_index[0], ar % N]).reshape(epad // CH, CH)
    dstp = jnp.concatenate([edge_index[1], N + (ar % 32)]).reshape(
        epad // CH, CH)

    degp = _deg_call(dstp, cpt)
    dega = degp[:N][:, None]
    degb = degp[NP:NP + N][:, None]

    g1, dis = _tc(_k2_body,
                  (jax.ShapeDtypeStruct((N, 128), _f32),
                   jax.ShapeDtypeStruct((N, 1), _f32)),
                  x, W1, dega, degb)

    acc1 = _gcn_call(g1, srcp, dstp, cpt, 128)
    t, asv, adv, adqv, q16 = _tc(
        _k4_body,
        (jax.ShapeDtypeStruct((N, 64), _f32),
         jax.ShapeDtypeStruct((N, 1), _f32),
         jax.ShapeDtypeStruct((N, 1), _f32),
         jax.ShapeDtypeStruct((N, 1), _f32),
         jax.ShapeDtypeStruct((16,), _f32)),
        acc1[:N], acc1[NP:NP + N], g1, dis, b1, W2, a_src, a_dst)

    zpad = jnp.zeros((NP - N,), _f32)
    asp = asv[:, 0]
    adp = jnp.concatenate([adv[:, 0], zpad])

    denp, acc2 = _gat_call(asp, adp, q16, t, srcp, dstp, cpt, 64)

    g3 = _tc(_k6_body, jax.ShapeDtypeStruct((N, 64), _f32),
             acc2[:N], acc2[NP:NP + N],
             denp[:N][:, None], denp[NP:NP + N][:, None],
             t, asv, adv, adqv, b2, W3, dis)

    acc3 = _gcn_call(g3, srcp, dstp, cpt, 64)

    out = _tc(_k8_body, jax.ShapeDtypeStruct((G, 1), _f32),
              acc3[:N], acc3[NP:NP + N], g3, dis, b3, batch[:, None], Wl,
              bl)
    return out


# speedup vs baseline: 1.5157x; 1.5157x over previous
import functools
from typing import Any

import jax
import jax.numpy as jnp
from jax import lax
from jax.experimental import pallas as pl
from jax.experimental.pallas import tpu as pltpu
from jax.experimental.pallas import tpu_sc as plsc

N = 10000
G = 16
NC = 2
NS = 16
NW = NC * NS
CH = 128
RPT = 640
NP = NS * RPT

_f32 = jnp.float32
_mesh = plsc.VectorSubcoreMesh(core_axis_name="c", subcore_axis_name="s")
_sc_params = pltpu.CompilerParams(use_tc_tiling_on_sc=False)


def _fill_1d(ref, n, val):
    def body(i, _):
        ref[pl.ds(i * 16, 16)] = jnp.full((16,), val, _f32)
        return _
    lax.fori_loop(0, n // 16, body, 0)


def _fill_rows(ref, nrows, d, val):
    def body(r, _):
        for c in range(d // 16):
            ref[r, pl.ds(c * 16, 16)] = jnp.full((16,), val, _f32)
        return _
    lax.fori_loop(0, nrows, body, 0)


def _wid(cid, sid):
    return sid * NC + cid


def _deg_body(cpt, dstp, out, deg_sh, dstv, ones_v, zer_v, sem):
    cid = lax.axis_index("c")
    sid = lax.axis_index("s")
    w = _wid(cid, sid)
    _fill_1d(zer_v, RPT, 0.0)
    _fill_1d(ones_v, CH, 1.0)
    pltpu.sync_copy(zer_v, deg_sh.at[pl.ds(sid * RPT, RPT)])
    plsc.subcore_barrier()
    pltpu.sync_copy(dstp.at[pl.ds(w * cpt, cpt)], dstv)

    def step(j, _):
        pltpu.sync_copy(ones_v, deg_sh.at[dstv.at[j]], add=True)
        return _
    lax.fori_loop(0, cpt, step, 0)
    plsc.subcore_barrier()
    pltpu.sync_copy(deg_sh.at[pl.ds(sid * RPT, RPT)],
                    out.at[pl.ds(cid * NP + sid * RPT, RPT)])


def _deg_call(dstp, cpt):
    return pl.kernel(
        functools.partial(_deg_body, cpt),
        out_type=jax.ShapeDtypeStruct((NC * NP,), _f32),
        mesh=_mesh,
        compiler_params=_sc_params,
        scratch_types=[
            pltpu.VMEM_SHARED((NP,), _f32),
            pltpu.VMEM((cpt, CH), jnp.int32),
            pltpu.VMEM((CH,), _f32),
            pltpu.VMEM((RPT,), _f32),
            pltpu.SemaphoreType.DMA,
        ],
    )(dstp)


_SS = 8


def _gcn_body(cpt, d, g, srcp, dstp, out, acc_sh, srcw, dstw, rows0, rows1,
              semi, sem0, sem1):
    cid = lax.axis_index("c")
    sid = lax.axis_index("s")
    w = _wid(cid, sid)
    nss = cpt // _SS
    _fill_rows(rows0, CH, d, 0.0)
    for r in range(RPT // CH):
        pltpu.sync_copy(rows0, acc_sh.at[pl.ds(sid * RPT + r * CH, CH)])
    plsc.subcore_barrier()
    pltpu.sync_copy(srcp.at[pl.ds(w * cpt, _SS)], srcw.at[0])
    pltpu.sync_copy(dstp.at[pl.ds(w * cpt, _SS)], dstw.at[0])

    def ss_body(p, _):
        b = p % 2
        nb = 1 - b
        off = w * cpt + jnp.minimum((p + 1) * _SS, cpt - _SS)
        pltpu.async_copy(srcp.at[pl.ds(off, _SS)], srcw.at[nb], semi)
        pltpu.async_copy(dstp.at[pl.ds(off, _SS)], dstw.at[nb], semi)
        bufs = (rows0, rows1)
        sems = (sem0, sem1)
        pltpu.async_copy(g.at[srcw.at[b, 0]], rows0, sem0)
        pltpu.async_copy(g.at[srcw.at[b, 1]], rows1, sem1)
        for k in range(_SS):
            rb, smb = bufs[k % 2], sems[k % 2]
            pltpu.make_async_copy(g.at[srcw.at[b, k]], rb, smb).wait()
            pltpu.sync_copy(rb, acc_sh.at[dstw.at[b, k]], add=True)
            if k + 2 < _SS:
                pltpu.async_copy(g.at[srcw.at[b, k + 2]], rb, smb)
        pltpu.make_async_copy(srcp.at[pl.ds(off, _SS)], srcw.at[nb],
                              semi).wait()
        pltpu.make_async_copy(dstp.at[pl.ds(off, _SS)], dstw.at[nb],
                              semi).wait()
        return _
    lax.fori_loop(0, nss, ss_body, 0)
    plsc.subcore_barrier()
    for r in range(RPT // CH):
        pltpu.sync_copy(acc_sh.at[pl.ds(sid * RPT + r * CH, CH)],
                        out.at[pl.ds(cid * NP + sid * RPT + r * CH, CH)])


def _gcn_call(g, srcp, dstp, cpt, d):
    return pl.kernel(
        functools.partial(_gcn_body, cpt, d),
        out_type=jax.ShapeDtypeStruct((NC * NP, d), _f32),
        mesh=_mesh,
        compiler_params=_sc_params,
        scratch_types=[
            pltpu.VMEM_SHARED((NP, d), _f32),
            pltpu.VMEM((2, _SS, CH), jnp.int32),
            pltpu.VMEM((2, _SS, CH), jnp.int32),
            pltpu.VMEM((CH, d), _f32),
            pltpu.VMEM((CH, d), _f32),
            pltpu.SemaphoreType.DMA,
            pltpu.SemaphoreType.DMA,
            pltpu.SemaphoreType.DMA,
        ],
    )(g, srcp, dstp)


def _gat_body(cpt, d, asp, adp, qh, t, srcp, dstp, den_out, acc_out,
              den_sh, acc_sh, srcv, dstv, asv0, asv1, adv0, adv1,
              exv0, exv1, rows0, rows1, qv, zer_v, sem0, sem1):
    cid = lax.axis_index("c")
    sid = lax.axis_index("s")
    w = _wid(cid, sid)
    _fill_1d(zer_v, RPT, 0.0)
    pltpu.sync_copy(zer_v, den_sh.at[pl.ds(sid * RPT, RPT)])
    _fill_rows(rows0, CH, d, 0.0)
    for r in range(RPT // CH):
        pltpu.sync_copy(rows0, acc_sh.at[pl.ds(sid * RPT + r * CH, CH)])
    plsc.subcore_barrier()
    pltpu.sync_copy(srcp.at[pl.ds(w * cpt, cpt)], srcv.at[pl.ds(0, cpt)])
    for rr in (cpt, cpt + 1):
        for c in range(CH // 16):
            srcv[rr, pl.ds(c * 16, 16)] = jnp.zeros((16,), jnp.int32)
            dstv[rr, pl.ds(c * 16, 16)] = jnp.zeros((16,), jnp.int32)
    pltpu.sync_copy(dstp.at[pl.ds(w * cpt, cpt)], dstv.at[pl.ds(0, cpt)])
    pltpu.sync_copy(qh, qv)
    qq = qv[...]

    def start(j, asv, adv, rows, sem):
        pltpu.async_copy(asp.at[srcv.at[j]], asv, sem)
        pltpu.async_copy(adp.at[dstv.at[j]], adv, sem)
        pltpu.async_copy(t.at[srcv.at[j]], rows, sem)

    def wait(j, asv, adv, rows, sem):
        pltpu.make_async_copy(asp.at[srcv.at[j]], asv, sem).wait()
        pltpu.make_async_copy(adp.at[dstv.at[j]], adv, sem).wait()
        pltpu.make_async_copy(t.at[srcv.at[j]], rows, sem).wait()

    def compute(j, asv, adv, exv, rows):
        for gidx in range(CH // 16):
            sl = pl.ds(gidx * 16, 16)
            u = asv[sl] + adv[sl]
            e = jnp.where(u > 0, u, 0.2 * u)
            ad = adv[sl]
            c_sh = jnp.where(ad > 0, ad, 0.2 * ad) + qq
            exv[sl] = jnp.exp(e - c_sh)
        pltpu.sync_copy(exv, den_sh.at[dstv.at[j]], add=True)

        def scale(gi, _c):
            exg = exv[pl.ds(gi * 16, 16)]
            for l in range(16):
                k = gi * 16 + l
                exb = jnp.full((16,), 1.0, _f32) * exg[l]
                for c in range(d // 16):
                    csl = pl.ds(c * 16, 16)
                    rows[k, csl] = rows[k, csl] * exb
            return _c
        lax.fori_loop(0, CH // 16, scale, 0)
        pltpu.sync_copy(rows, acc_sh.at[dstv.at[j]], add=True)

    start(0, asv0, adv0, rows0, sem0)

    def step(p, _):
        j0 = p * 2
        start(j0 + 1, asv1, adv1, rows1, sem1)
        wait(j0, asv0, adv0, rows0, sem0)
        compute(j0, asv0, adv0, exv0, rows0)
        start(j0 + 2, asv0, adv0, rows0, sem0)
        wait(j0 + 1, asv1, adv1, rows1, sem1)
        compute(j0 + 1, asv1, adv1, exv1, rows1)
        return _
    lax.fori_loop(0, cpt // 2, step, 0)
    wait(cpt, asv0, adv0, rows0, sem0)
    plsc.subcore_barrier()
    pltpu.sync_copy(den_sh.at[pl.ds(sid * RPT, RPT)],
                    den_out.at[pl.ds(cid * NP + sid * RPT, RPT)])
    for r in range(RPT // CH):
        pltpu.sync_copy(acc_sh.at[pl.ds(sid * RPT + r * CH, CH)],
                        acc_out.at[pl.ds(cid * NP + sid * RPT + r * CH, CH)])


def _gat_call(asp, adp, qh, t, srcp, dstp, cpt, d):
    return pl.kernel(
        functools.partial(_gat_body, cpt, d),
        out_type=(jax.ShapeDtypeStruct((NC * NP,), _f32),
                  jax.ShapeDtypeStruct((NC * NP, d), _f32)),
        mesh=_mesh,
        compiler_params=_sc_params,
        scratch_types=[
            pltpu.VMEM_SHARED((NP,), _f32),
            pltpu.VMEM_SHARED((NP, d), _f32),
            pltpu.VMEM((cpt + 2, CH), jnp.int32),
            pltpu.VMEM((cpt + 2, CH), jnp.int32),
            pltpu.VMEM((CH,), _f32),
            pltpu.VMEM((CH,), _f32),
            pltpu.VMEM((CH,), _f32),
            pltpu.VMEM((CH,), _f32),
            pltpu.VMEM((CH,), _f32),
            pltpu.VMEM((CH,), _f32),
            pltpu.VMEM((CH, d), _f32),
            pltpu.VMEM((CH, d), _f32),
            pltpu.VMEM((16,), _f32),
            pltpu.VMEM((RPT,), _f32),
            pltpu.SemaphoreType.DMA,
            pltpu.SemaphoreType.DMA,
        ],
    )(asp, adp, qh, t, srcp, dstp)


def _leaky01(v):
    return jnp.where(v > 0, v, 0.01 * v)


def _leaky20(v):
    return jnp.where(v > 0, v, 0.2 * v)


def _k2_body(x_ref, w1_ref, dega_ref, degb_ref, g1_ref, dis_ref):
    deg = dega_ref[...] + degb_ref[...] + 1.0
    dis = lax.rsqrt(deg)
    p1 = jnp.dot(x_ref[...], w1_ref[...], preferred_element_type=_f32)
    g1_ref[...] = dis * p1
    dis_ref[...] = dis


def _k4_body(acc_a, acc_b, g1, dis, b1, w2, a_s, a_d, t_o, as_o, ad_o,
             adq_o, q_o):
    h1 = _leaky01(dis[...] * (acc_a[...] + acc_b[...] + g1[...])
                  + b1[...][None, :])
    t = jnp.dot(h1, w2[...], preferred_element_type=_f32)
    asv = jnp.dot(t, a_s[...][:, None], preferred_element_type=_f32)
    adv = jnp.dot(t, a_d[...][:, None], preferred_element_type=_f32)
    m = jnp.max(asv)
    q = jnp.maximum(m, 0.2 * m)
    t_o[...] = t
    as_o[...] = asv
    ad_o[...] = adv
    adq_o[...] = _leaky20(adv) + q
    q_o[...] = jnp.full((16,), 1.0, _f32) * q


def _k6_body(acc_a, acc_b, den_a, den_b, t, asv, adv, adqv, b2, w3, dis,
             g3_o):
    exs = jnp.exp(_leaky20(asv[...] + adv[...]) - adqv[...])
    den = jnp.maximum(den_a[...] + den_b[...] + exs, 1e-16)
    num = acc_a[...] + acc_b[...] + exs * t[...]
    h2 = _leaky01(num / den + b2[...][None, :])
    p3 = jnp.dot(h2, w3[...], preferred_element_type=_f32)
    g3_o[...] = dis[...] * p3


def _k8_body(acc_a, acc_b, g3, dis, b3, batch, wl, bl, out):
    h3 = _leaky01(dis[...] * (acc_a[...] + acc_b[...] + g3[...])
                  + b3[...][None, :])
    oh = (batch[...] == lax.broadcasted_iota(jnp.int32, (N, G), 1))
    oh = oh.astype(_f32)
    sums = lax.dot_general(oh, h3, (((0,), (0,)), ((), ())),
                           preferred_element_type=_f32)
    cnt = jnp.sum(oh, axis=0)[:, None]
    pooled = sums / jnp.maximum(cnt, 1.0)
    out[...] = jnp.dot(pooled, wl[...], preferred_element_type=_f32) \
        + bl[...][None, :]


def _tc(body, out_shape, *args):
    return pl.pallas_call(body, out_shape=out_shape)(*args)


def kernel(x, edge_index, batch, W1, b1, W2, a_src, a_dst, b2, W3, b3, Wl,
           bl):
    e = edge_index.shape[1]
    cpt = -(-e // (CH * NW))
    cpt = -(-cpt // 8) * 8
    epad = cpt * NW * CH
    pad = epad - e
    ar = jnp.arange(pad, dtype=jnp.int32)
    srcp = jnp.concatenate([edge_index[0], ar % N]).reshape(epad // CH, CH)
    dstp = jnp.concatenate([edge_index[1], N + (ar % 32)]).reshape(
        epad // CH, CH)

    degp = _deg_call(dstp, cpt)
    dega = degp[:N][:, None]
    degb = degp[NP:NP + N][:, None]

    g1, dis = _tc(_k2_body,
                  (jax.ShapeDtypeStruct((N, 128), _f32),
                   jax.ShapeDtypeStruct((N, 1), _f32)),
                  x, W1, dega, degb)

    acc1 = _gcn_call(g1, srcp, dstp, cpt, 128)
    t, asv, adv, adqv, q16 = _tc(
        _k4_body,
        (jax.ShapeDtypeStruct((N, 64), _f32),
         jax.ShapeDtypeStruct((N, 1), _f32),
         jax.ShapeDtypeStruct((N, 1), _f32),
         jax.ShapeDtypeStruct((N, 1), _f32),
         jax.ShapeDtypeStruct((16,), _f32)),
        acc1[:N], acc1[NP:NP + N], g1, dis, b1, W2, a_src, a_dst)

    zpad = jnp.zeros((NP - N,), _f32)
    asp = asv[:, 0]
    adp = jnp.concatenate([adv[:, 0], zpad])

    denp, acc2 = _gat_call(asp, adp, q16, t, srcp, dstp, cpt, 64)

    g3 = _tc(_k6_body, jax.ShapeDtypeStruct((N, 64), _f32),
             acc2[:N], acc2[NP:NP + N],
             denp[:N][:, None], denp[NP:NP + N][:, None],
             t, asv, adv, adqv, b2, W3, dis)

    acc3 = _gcn_call(g3, srcp, dstp, cpt, 64)

    out = _tc(_k8_body, jax.ShapeDtypeStruct((G, 1), _f32),
              acc3[:N], acc3[NP:NP + N], g3, dis, b3, batch[:, None], Wl,
              bl)
    return out


# speedup vs baseline: 1.5626x; 1.0310x over previous
import functools
from typing import Any

import jax
import jax.numpy as jnp
from jax import lax
from jax.experimental import pallas as pl
from jax.experimental.pallas import tpu as pltpu
from jax.experimental.pallas import tpu_sc as plsc

N = 10000
G = 16
NC = 2
NS = 16
NW = NC * NS
CH = 128
RPT = 640
NP = NS * RPT

_f32 = jnp.float32
_mesh = plsc.VectorSubcoreMesh(core_axis_name="c", subcore_axis_name="s")
_sc_params = pltpu.CompilerParams(use_tc_tiling_on_sc=False)


def _fill_1d(ref, n, val):
    def body(i, _):
        ref[pl.ds(i * 16, 16)] = jnp.full((16,), val, _f32)
        return _
    lax.fori_loop(0, n // 16, body, 0)


def _fill_rows(ref, nrows, d, val):
    def body(r, _):
        for c in range(d // 16):
            ref[r, pl.ds(c * 16, 16)] = jnp.full((16,), val, _f32)
        return _
    lax.fori_loop(0, nrows, body, 0)


def _wid(cid, sid):
    return sid * NC + cid


def _deg_body(cpt, dstp, out, deg_sh, dstv, ones_v, zer_v, sem):
    cid = lax.axis_index("c")
    sid = lax.axis_index("s")
    w = _wid(cid, sid)
    _fill_1d(zer_v, RPT, 0.0)
    _fill_1d(ones_v, CH, 1.0)
    pltpu.sync_copy(zer_v, deg_sh.at[pl.ds(sid * RPT, RPT)])
    plsc.subcore_barrier()
    pltpu.sync_copy(dstp.at[pl.ds(w * cpt, cpt)], dstv)

    def step(j, _):
        pltpu.sync_copy(ones_v, deg_sh.at[dstv.at[j]], add=True)
        return _
    lax.fori_loop(0, cpt, step, 0)
    plsc.subcore_barrier()
    pltpu.sync_copy(deg_sh.at[pl.ds(sid * RPT, RPT)],
                    out.at[pl.ds(cid * NP + sid * RPT, RPT)])


def _deg_call(dstp, cpt):
    return pl.kernel(
        functools.partial(_deg_body, cpt),
        out_type=jax.ShapeDtypeStruct((NC * NP,), _f32),
        mesh=_mesh,
        compiler_params=_sc_params,
        scratch_types=[
            pltpu.VMEM_SHARED((NP,), _f32),
            pltpu.VMEM((cpt, CH), jnp.int32),
            pltpu.VMEM((CH,), _f32),
            pltpu.VMEM((RPT,), _f32),
            pltpu.SemaphoreType.DMA,
        ],
    )(dstp)


_SS = 8


def _gcn_body(cpt, d, g, srcp, dstp, out, acc_sh, srcw, dstw, rows0, rows1,
              semi, sem0, sem1):
    cid = lax.axis_index("c")
    sid = lax.axis_index("s")
    w = _wid(cid, sid)
    nss = cpt // _SS
    _fill_rows(rows0, CH, d, 0.0)
    for r in range(RPT // CH):
        pltpu.sync_copy(rows0, acc_sh.at[pl.ds(sid * RPT + r * CH, CH)])
    plsc.subcore_barrier()
    pltpu.sync_copy(srcp.at[pl.ds(w * cpt, _SS)], srcw.at[0])
    pltpu.sync_copy(dstp.at[pl.ds(w * cpt, _SS)], dstw.at[0])

    def ss_body(p, _):
        b = p % 2
        nb = 1 - b
        off = w * cpt + jnp.minimum((p + 1) * _SS, cpt - _SS)
        pltpu.async_copy(srcp.at[pl.ds(off, _SS)], srcw.at[nb], semi)
        pltpu.async_copy(dstp.at[pl.ds(off, _SS)], dstw.at[nb], semi)
        bufs = (rows0, rows1)
        sems = (sem0, sem1)
        pltpu.async_copy(g.at[srcw.at[b, 0]], rows0, sem0)
        pltpu.async_copy(g.at[srcw.at[b, 1]], rows1, sem1)
        for k in range(_SS):
            rb, smb = bufs[k % 2], sems[k % 2]
            pltpu.make_async_copy(g.at[srcw.at[b, k]], rb, smb).wait()
            pltpu.sync_copy(rb, acc_sh.at[dstw.at[b, k]], add=True)
            if k + 2 < _SS:
                pltpu.async_copy(g.at[srcw.at[b, k + 2]], rb, smb)
        pltpu.make_async_copy(srcp.at[pl.ds(off, _SS)], srcw.at[nb],
                              semi).wait()
        pltpu.make_async_copy(dstp.at[pl.ds(off, _SS)], dstw.at[nb],
                              semi).wait()
        return _
    lax.fori_loop(0, nss, ss_body, 0)
    plsc.subcore_barrier()
    for r in range(RPT // CH):
        pltpu.sync_copy(acc_sh.at[pl.ds(sid * RPT + r * CH, CH)],
                        out.at[pl.ds(cid * NP + sid * RPT + r * CH, CH)])


def _gcn_call(g, srcp, dstp, cpt, d):
    return pl.kernel(
        functools.partial(_gcn_body, cpt, d),
        out_type=jax.ShapeDtypeStruct((NC * NP, d), _f32),
        mesh=_mesh,
        compiler_params=_sc_params,
        scratch_types=[
            pltpu.VMEM_SHARED((NP, d), _f32),
            pltpu.VMEM((2, _SS, CH), jnp.int32),
            pltpu.VMEM((2, _SS, CH), jnp.int32),
            pltpu.VMEM((CH, d), _f32),
            pltpu.VMEM((CH, d), _f32),
            pltpu.SemaphoreType.DMA,
            pltpu.SemaphoreType.DMA,
            pltpu.SemaphoreType.DMA,
        ],
    )(g, srcp, dstp)


def _gat_body(cpt, d, asp, adp, qh, t, srcp, dstp, den_out, acc_out,
              den_sh, acc_sh, srcv, dstv, asv0, asv1, adv0, adv1,
              exv0, exv1, rows0, rows1, qv, zer_v, sem0, sem1, semd):
    cid = lax.axis_index("c")
    sid = lax.axis_index("s")
    w = _wid(cid, sid)
    _fill_1d(zer_v, RPT, 0.0)
    pltpu.sync_copy(zer_v, den_sh.at[pl.ds(sid * RPT, RPT)])
    _fill_rows(rows0, CH, d, 0.0)
    for r in range(RPT // CH):
        pltpu.sync_copy(rows0, acc_sh.at[pl.ds(sid * RPT + r * CH, CH)])
    plsc.subcore_barrier()
    pltpu.sync_copy(srcp.at[pl.ds(w * cpt, cpt)], srcv.at[pl.ds(0, cpt)])
    for rr in (cpt, cpt + 1):
        for c in range(CH // 16):
            srcv[rr, pl.ds(c * 16, 16)] = jnp.zeros((16,), jnp.int32)
            dstv[rr, pl.ds(c * 16, 16)] = jnp.zeros((16,), jnp.int32)
    pltpu.sync_copy(dstp.at[pl.ds(w * cpt, cpt)], dstv.at[pl.ds(0, cpt)])
    pltpu.sync_copy(qh, qv)
    qq = qv[...]

    def start(j, asv, adv, rows, sem):
        pltpu.async_copy(asp.at[srcv.at[j]], asv, sem)
        pltpu.async_copy(adp.at[dstv.at[j]], adv, sem)
        pltpu.async_copy(t.at[srcv.at[j]], rows, sem)

    def wait(j, asv, adv, rows, sem):
        pltpu.make_async_copy(asp.at[srcv.at[j]], asv, sem).wait()
        pltpu.make_async_copy(adp.at[dstv.at[j]], adv, sem).wait()
        pltpu.make_async_copy(t.at[srcv.at[j]], rows, sem).wait()

    def compute(j, asv, adv, exv, rows):
        pltpu.make_async_copy(exv, den_sh.at[dstv.at[j]], semd).wait()
        for gidx in range(CH // 16):
            sl = pl.ds(gidx * 16, 16)
            u = asv[sl] + adv[sl]
            e = jnp.where(u > 0, u, 0.2 * u)
            ad = adv[sl]
            c_sh = jnp.where(ad > 0, ad, 0.2 * ad) + qq
            exv[sl] = jnp.exp(e - c_sh)
        pltpu.async_copy(exv, den_sh.at[dstv.at[j]], semd, add=True)

        def scale(gi, _c):
            exg = exv[pl.ds(gi * 16, 16)]
            for l in range(16):
                k = gi * 16 + l
                exb = jnp.full((16,), 1.0, _f32) * exg[l]
                for c in range(d // 16):
                    csl = pl.ds(c * 16, 16)
                    rows[k, csl] = rows[k, csl] * exb
            return _c
        lax.fori_loop(0, CH // 16, scale, 0)
        pltpu.sync_copy(rows, acc_sh.at[dstv.at[j]], add=True)

    _fill_1d(exv0, CH, 0.0)
    _fill_1d(exv1, CH, 0.0)
    pltpu.async_copy(exv0, den_sh.at[dstv.at[cpt]], semd, add=True)
    pltpu.async_copy(exv1, den_sh.at[dstv.at[cpt]], semd, add=True)
    start(0, asv0, adv0, rows0, sem0)

    def step(p, _):
        j0 = p * 2
        start(j0 + 1, asv1, adv1, rows1, sem1)
        wait(j0, asv0, adv0, rows0, sem0)
        compute(j0, asv0, adv0, exv0, rows0)
        start(j0 + 2, asv0, adv0, rows0, sem0)
        wait(j0 + 1, asv1, adv1, rows1, sem1)
        compute(j0 + 1, asv1, adv1, exv1, rows1)
        return _
    lax.fori_loop(0, cpt // 2, step, 0)
    wait(cpt, asv0, adv0, rows0, sem0)
    pltpu.make_async_copy(exv0, den_sh.at[dstv.at[cpt]], semd).wait()
    pltpu.make_async_copy(exv1, den_sh.at[dstv.at[cpt]], semd).wait()
    plsc.subcore_barrier()
    pltpu.sync_copy(den_sh.at[pl.ds(sid * RPT, RPT)],
                    den_out.at[pl.ds(cid * NP + sid * RPT, RPT)])
    for r in range(RPT // CH):
        pltpu.sync_copy(acc_sh.at[pl.ds(sid * RPT + r * CH, CH)],
                        acc_out.at[pl.ds(cid * NP + sid * RPT + r * CH, CH)])


def _gat_call(asp, adp, qh, t, srcp, dstp, cpt, d):
    return pl.kernel(
        functools.partial(_gat_body, cpt, d),
        out_type=(jax.ShapeDtypeStruct((NC * NP,), _f32),
                  jax.ShapeDtypeStruct((NC * NP, d), _f32)),
        mesh=_mesh,
        compiler_params=_sc_params,
        scratch_types=[
            pltpu.VMEM_SHARED((NP,), _f32),
            pltpu.VMEM_SHARED((NP, d), _f32),
            pltpu.VMEM((cpt + 2, CH), jnp.int32),
            pltpu.VMEM((cpt + 2, CH), jnp.int32),
            pltpu.VMEM((CH,), _f32),
            pltpu.VMEM((CH,), _f32),
            pltpu.VMEM((CH,), _f32),
            pltpu.VMEM((CH,), _f32),
            pltpu.VMEM((CH,), _f32),
            pltpu.VMEM((CH,), _f32),
            pltpu.VMEM((CH, d), _f32),
            pltpu.VMEM((CH, d), _f32),
            pltpu.VMEM((16,), _f32),
            pltpu.VMEM((RPT,), _f32),
            pltpu.SemaphoreType.DMA,
            pltpu.SemaphoreType.DMA,
            pltpu.SemaphoreType.DMA,
        ],
    )(asp, adp, qh, t, srcp, dstp)


def _leaky01(v):
    return jnp.where(v > 0, v, 0.01 * v)


def _leaky20(v):
    return jnp.where(v > 0, v, 0.2 * v)


def _k2_body(x_ref, w1_ref, dega_ref, degb_ref, g1_ref, dis_ref):
    deg = dega_ref[...] + degb_ref[...] + 1.0
    dis = lax.rsqrt(deg)
    p1 = jnp.dot(x_ref[...], w1_ref[...], preferred_element_type=_f32)
    g1_ref[...] = dis * p1
    dis_ref[...] = dis


def _k4_body(acc, g1, dis, b1, w2, a_s, a_d, t_o, as_o, ad_o,
             adq_o, q_o):
    h1 = _leaky01(dis[...] * (acc[0:N, :] + acc[NP:NP + N, :] + g1[...])
                  + b1[...][None, :])
    t = jnp.dot(h1, w2[...], preferred_element_type=_f32)
    asv = jnp.dot(t, a_s[...][:, None], preferred_element_type=_f32)
    adv = jnp.dot(t, a_d[...][:, None], preferred_element_type=_f32)
    m = jnp.max(asv)
    q = jnp.maximum(m, 0.2 * m)
    t_o[...] = t
    as_o[...] = asv
    ad_o[...] = adv
    adq_o[...] = _leaky20(adv) + q
    q_o[...] = jnp.full((16,), 1.0, _f32) * q


def _k6_body(acc_a, acc_b, den_a, den_b, t, asv, adv, adqv, b2, w3, dis,
             g3_o):
    exs = jnp.exp(_leaky20(asv[...] + adv[...]) - adqv[...])
    den = jnp.maximum(den_a[...] + den_b[...] + exs, 1e-16)
    num = acc_a[...] + acc_b[...] + exs * t[...]
    h2 = _leaky01(num / den + b2[...][None, :])
    p3 = jnp.dot(h2, w3[...], preferred_element_type=_f32)
    g3_o[...] = dis[...] * p3


def _k8_body(acc, g3, dis, b3, batch, wl, bl, out):
    h3 = _leaky01(dis[...] * (acc[0:N, :] + acc[NP:NP + N, :] + g3[...])
                  + b3[...][None, :])
    oh = (batch[...] == lax.broadcasted_iota(jnp.int32, (N, G), 1))
    oh = oh.astype(_f32)
    sums = lax.dot_general(oh, h3, (((0,), (0,)), ((), ())),
                           preferred_element_type=_f32)
    cnt = jnp.sum(oh, axis=0)[:, None]
    pooled = sums / jnp.maximum(cnt, 1.0)
    out[...] = jnp.dot(pooled, wl[...], preferred_element_type=_f32) \
        + bl[...][None, :]


def _tc(body, out_shape, *args):
    return pl.pallas_call(body, out_shape=out_shape)(*args)


def kernel(x, edge_index, batch, W1, b1, W2, a_src, a_dst, b2, W3, b3, Wl,
           bl):
    e = edge_index.shape[1]
    cpt = -(-e // (CH * NW))
    cpt = -(-cpt // 8) * 8
    epad = cpt * NW * CH
    pad = epad - e
    ar = jnp.arange(pad, dtype=jnp.int32)
    srcp = jnp.concatenate([edge_index[0], ar % N]).reshape(epad // CH, CH)
    dstp = jnp.concatenate([edge_index[1], N + (ar % 32)]).reshape(
        epad // CH, CH)

    degp = _deg_call(dstp, cpt)
    dega = degp[:N][:, None]
    degb = degp[NP:NP + N][:, None]

    g1, dis = _tc(_k2_body,
                  (jax.ShapeDtypeStruct((N, 128), _f32),
                   jax.ShapeDtypeStruct((N, 1), _f32)),
                  x, W1, dega, degb)

    acc1 = _gcn_call(g1, srcp, dstp, cpt, 128)
    t, asv, adv, adqv, q16 = _tc(
        _k4_body,
        (jax.ShapeDtypeStruct((N, 64), _f32),
         jax.ShapeDtypeStruct((N, 1), _f32),
         jax.ShapeDtypeStruct((N, 1), _f32),
         jax.ShapeDtypeStruct((N, 1), _f32),
         jax.ShapeDtypeStruct((16,), _f32)),
        acc1, g1, dis, b1, W2, a_src, a_dst)

    zpad = jnp.zeros((NP - N,), _f32)
    asp = asv[:, 0]
    adp = jnp.concatenate([adv[:, 0], zpad])

    denp, acc2 = _gat_call(asp, adp, q16, t, srcp, dstp, cpt, 64)

    g3 = _tc(_k6_body, jax.ShapeDtypeStruct((N, 64), _f32),
             acc2[:N], acc2[NP:NP + N],
             denp[:N][:, None], denp[NP:NP + N][:, None],
             t, asv, adv, adqv, b2, W3, dis)

    acc3 = _gcn_call(g3, srcp, dstp, cpt, 64)

    out = _tc(_k8_body, jax.ShapeDtypeStruct((G, 1), _f32),
              acc3, g3, dis, b3, batch[:, None], Wl, bl)
    return out


# speedup vs baseline: 1.5859x; 1.0149x over previous
import functools
from typing import Any

import jax
import jax.numpy as jnp
from jax import lax
from jax.experimental import pallas as pl
from jax.experimental.pallas import tpu as pltpu
from jax.experimental.pallas import tpu_sc as plsc

N = 10000
G = 16
NC = 2
NS = 16
NW = NC * NS
CH = 128
RPT = 640
NP = NS * RPT

_f32 = jnp.float32
_mesh = plsc.VectorSubcoreMesh(core_axis_name="c", subcore_axis_name="s")
_sc_params = pltpu.CompilerParams(use_tc_tiling_on_sc=False)


def _fill_1d(ref, n, val):
    def body(i, _):
        ref[pl.ds(i * 16, 16)] = jnp.full((16,), val, _f32)
        return _
    lax.fori_loop(0, n // 16, body, 0)


def _fill_rows(ref, nrows, d, val):
    def body(r, _):
        for c in range(d // 16):
            ref[r, pl.ds(c * 16, 16)] = jnp.full((16,), val, _f32)
        return _
    lax.fori_loop(0, nrows, body, 0)


def _wid(cid, sid):
    return sid * NC + cid


def _deg_body(cpt, dstp, out, deg_sh, dstv, ones_v, zer_v, sem):
    cid = lax.axis_index("c")
    sid = lax.axis_index("s")
    w = _wid(cid, sid)
    _fill_1d(zer_v, RPT, 0.0)
    _fill_1d(ones_v, CH, 1.0)
    pltpu.sync_copy(zer_v, deg_sh.at[pl.ds(sid * RPT, RPT)])
    plsc.subcore_barrier()
    pltpu.sync_copy(dstp.at[pl.ds(w * cpt, cpt)], dstv)

    def step(j, _):
        pltpu.sync_copy(ones_v, deg_sh.at[dstv.at[j]], add=True)
        return _
    lax.fori_loop(0, cpt, step, 0)
    plsc.subcore_barrier()
    pltpu.sync_copy(deg_sh.at[pl.ds(sid * RPT, RPT)],
                    out.at[pl.ds(cid * NP + sid * RPT, RPT)])


def _deg_call(dstp, cpt):
    return pl.kernel(
        functools.partial(_deg_body, cpt),
        out_type=jax.ShapeDtypeStruct((NC * NP,), _f32),
        mesh=_mesh,
        compiler_params=_sc_params,
        scratch_types=[
            pltpu.VMEM_SHARED((NP,), _f32),
            pltpu.VMEM((cpt, CH), jnp.int32),
            pltpu.VMEM((CH,), _f32),
            pltpu.VMEM((RPT,), _f32),
            pltpu.SemaphoreType.DMA,
        ],
    )(dstp)


_SS = 16


def _gcn_body(cpt, d, g, srcp, dstp, out, acc_sh, srcw, dstw, rows0, rows1,
              semi, sem0, sem1):
    cid = lax.axis_index("c")
    sid = lax.axis_index("s")
    w = _wid(cid, sid)
    nss = cpt // _SS
    _fill_rows(rows0, CH, d, 0.0)
    for r in range(RPT // CH):
        pltpu.sync_copy(rows0, acc_sh.at[pl.ds(sid * RPT + r * CH, CH)])
    plsc.subcore_barrier()
    pltpu.sync_copy(srcp.at[pl.ds(w * cpt, _SS)], srcw.at[0])
    pltpu.sync_copy(dstp.at[pl.ds(w * cpt, _SS)], dstw.at[0])

    def ss_body(p, _):
        b = p % 2
        nb = 1 - b
        off = w * cpt + jnp.minimum((p + 1) * _SS, cpt - _SS)
        pltpu.async_copy(srcp.at[pl.ds(off, _SS)], srcw.at[nb], semi)
        pltpu.async_copy(dstp.at[pl.ds(off, _SS)], dstw.at[nb], semi)
        bufs = (rows0, rows1)
        sems = (sem0, sem1)
        pltpu.async_copy(g.at[srcw.at[b, 0]], rows0, sem0)
        pltpu.async_copy(g.at[srcw.at[b, 1]], rows1, sem1)
        for k in range(_SS):
            rb, smb = bufs[k % 2], sems[k % 2]
            pltpu.make_async_copy(g.at[srcw.at[b, k]], rb, smb).wait()
            pltpu.sync_copy(rb, acc_sh.at[dstw.at[b, k]], add=True)
            if k + 2 < _SS:
                pltpu.async_copy(g.at[srcw.at[b, k + 2]], rb, smb)
        pltpu.make_async_copy(srcp.at[pl.ds(off, _SS)], srcw.at[nb],
                              semi).wait()
        pltpu.make_async_copy(dstp.at[pl.ds(off, _SS)], dstw.at[nb],
                              semi).wait()
        return _
    lax.fori_loop(0, nss, ss_body, 0)
    plsc.subcore_barrier()
    for r in range(RPT // CH):
        pltpu.sync_copy(acc_sh.at[pl.ds(sid * RPT + r * CH, CH)],
                        out.at[pl.ds(cid * NP + sid * RPT + r * CH, CH)])


def _gcn_call(g, srcp, dstp, cpt, d):
    return pl.kernel(
        functools.partial(_gcn_body, cpt, d),
        out_type=jax.ShapeDtypeStruct((NC * NP, d), _f32),
        mesh=_mesh,
        compiler_params=_sc_params,
        scratch_types=[
            pltpu.VMEM_SHARED((NP, d), _f32),
            pltpu.VMEM((2, _SS, CH), jnp.int32),
            pltpu.VMEM((2, _SS, CH), jnp.int32),
            pltpu.VMEM((CH, d), _f32),
            pltpu.VMEM((CH, d), _f32),
            pltpu.SemaphoreType.DMA,
            pltpu.SemaphoreType.DMA,
            pltpu.SemaphoreType.DMA,
        ],
    )(g, srcp, dstp)


def _gat_body(cpt, d, asp, adp, qh, t, srcp, dstp, den_out, acc_out,
              den_sh, acc_sh, srcv, dstv, asv0, asv1, adv0, adv1,
              exv0, exv1, rows0, rows1, qv, zer_v, sem0, sem1, semd):
    cid = lax.axis_index("c")
    sid = lax.axis_index("s")
    w = _wid(cid, sid)
    _fill_1d(zer_v, RPT, 0.0)
    pltpu.sync_copy(zer_v, den_sh.at[pl.ds(sid * RPT, RPT)])
    _fill_rows(rows0, CH, d, 0.0)
    for r in range(RPT // CH):
        pltpu.sync_copy(rows0, acc_sh.at[pl.ds(sid * RPT + r * CH, CH)])
    plsc.subcore_barrier()
    pltpu.sync_copy(srcp.at[pl.ds(w * cpt, cpt)], srcv.at[pl.ds(0, cpt)])
    for rr in (cpt, cpt + 1):
        for c in range(CH // 16):
            srcv[rr, pl.ds(c * 16, 16)] = jnp.zeros((16,), jnp.int32)
            dstv[rr, pl.ds(c * 16, 16)] = jnp.zeros((16,), jnp.int32)
    pltpu.sync_copy(dstp.at[pl.ds(w * cpt, cpt)], dstv.at[pl.ds(0, cpt)])
    pltpu.sync_copy(qh, qv)
    qq = qv[...]

    def start(j, asv, adv, rows, sem):
        pltpu.async_copy(asp.at[srcv.at[j]], asv, sem)
        pltpu.async_copy(adp.at[dstv.at[j]], adv, sem)
        pltpu.async_copy(t.at[srcv.at[j]], rows, sem)

    def wait(j, asv, adv, rows, sem):
        pltpu.make_async_copy(asp.at[srcv.at[j]], asv, sem).wait()
        pltpu.make_async_copy(adp.at[dstv.at[j]], adv, sem).wait()
        pltpu.make_async_copy(t.at[srcv.at[j]], rows, sem).wait()

    def compute(j, asv, adv, exv, rows):
        pltpu.make_async_copy(exv, den_sh.at[dstv.at[j]], semd).wait()
        for gidx in range(CH // 16):
            sl = pl.ds(gidx * 16, 16)
            u = asv[sl] + adv[sl]
            e = jnp.where(u > 0, u, 0.2 * u)
            ad = adv[sl]
            c_sh = jnp.where(ad > 0, ad, 0.2 * ad) + qq
            exv[sl] = jnp.exp(e - c_sh)
        pltpu.async_copy(exv, den_sh.at[dstv.at[j]], semd, add=True)

        def scale(gi, _c):
            exg = exv[pl.ds(gi * 16, 16)]
            for l in range(16):
                k = gi * 16 + l
                exb = jnp.full((16,), 1.0, _f32) * exg[l]
                for c in range(d // 16):
                    csl = pl.ds(c * 16, 16)
                    rows[k, csl] = rows[k, csl] * exb
            return _c
        lax.fori_loop(0, CH // 16, scale, 0)
        pltpu.sync_copy(rows, acc_sh.at[dstv.at[j]], add=True)

    _fill_1d(exv0, CH, 0.0)
    _fill_1d(exv1, CH, 0.0)
    pltpu.async_copy(exv0, den_sh.at[dstv.at[cpt]], semd, add=True)
    pltpu.async_copy(exv1, den_sh.at[dstv.at[cpt]], semd, add=True)
    start(0, asv0, adv0, rows0, sem0)

    def step(p, _):
        j0 = p * 2
        start(j0 + 1, asv1, adv1, rows1, sem1)
        wait(j0, asv0, adv0, rows0, sem0)
        compute(j0, asv0, adv0, exv0, rows0)
        start(j0 + 2, asv0, adv0, rows0, sem0)
        wait(j0 + 1, asv1, adv1, rows1, sem1)
        compute(j0 + 1, asv1, adv1, exv1, rows1)
        return _
    lax.fori_loop(0, cpt // 2, step, 0)
    wait(cpt, asv0, adv0, rows0, sem0)
    pltpu.make_async_copy(exv0, den_sh.at[dstv.at[cpt]], semd).wait()
    pltpu.make_async_copy(exv1, den_sh.at[dstv.at[cpt]], semd).wait()
    plsc.subcore_barrier()
    pltpu.sync_copy(den_sh.at[pl.ds(sid * RPT, RPT)],
                    den_out.at[pl.ds(cid * NP + sid * RPT, RPT)])
    for r in range(RPT // CH):
        pltpu.sync_copy(acc_sh.at[pl.ds(sid * RPT + r * CH, CH)],
                        acc_out.at[pl.ds(cid * NP + sid * RPT + r * CH, CH)])


def _gat_call(asp, adp, qh, t, srcp, dstp, cpt, d):
    return pl.kernel(
        functools.partial(_gat_body, cpt, d),
        out_type=(jax.ShapeDtypeStruct((NC * NP,), _f32),
                  jax.ShapeDtypeStruct((NC * NP, d), _f32)),
        mesh=_mesh,
        compiler_params=_sc_params,
        scratch_types=[
            pltpu.VMEM_SHARED((NP,), _f32),
            pltpu.VMEM_SHARED((NP, d), _f32),
            pltpu.VMEM((cpt + 2, CH), jnp.int32),
            pltpu.VMEM((cpt + 2, CH), jnp.int32),
            pltpu.VMEM((CH,), _f32),
            pltpu.VMEM((CH,), _f32),
            pltpu.VMEM((CH,), _f32),
            pltpu.VMEM((CH,), _f32),
            pltpu.VMEM((CH,), _f32),
            pltpu.VMEM((CH,), _f32),
            pltpu.VMEM((CH, d), _f32),
            pltpu.VMEM((CH, d), _f32),
            pltpu.VMEM((16,), _f32),
            pltpu.VMEM((RPT,), _f32),
            pltpu.SemaphoreType.DMA,
            pltpu.SemaphoreType.DMA,
            pltpu.SemaphoreType.DMA,
        ],
    )(asp, adp, qh, t, srcp, dstp)


def _leaky01(v):
    return jnp.where(v > 0, v, 0.01 * v)


def _leaky20(v):
    return jnp.where(v > 0, v, 0.2 * v)


def _k2_body(x_ref, w1_ref, dega_ref, degb_ref, g1_ref, dis_ref):
    deg = dega_ref[...] + degb_ref[...] + 1.0
    dis = lax.rsqrt(deg)
    p1 = jnp.dot(x_ref[...], w1_ref[...], preferred_element_type=_f32)
    g1_ref[...] = dis * p1
    dis_ref[...] = dis


def _k4_body(acc, g1, dis, b1, w2, a_s, a_d, t_o, as_o, ad_o,
             adq_o, q_o):
    h1 = _leaky01(dis[...] * (acc[0:N, :] + acc[NP:NP + N, :] + g1[...])
                  + b1[...][None, :])
    t = jnp.dot(h1, w2[...], preferred_element_type=_f32)
    asv = jnp.dot(t, a_s[...][:, None], preferred_element_type=_f32)
    adv = jnp.dot(t, a_d[...][:, None], preferred_element_type=_f32)
    m = jnp.max(asv)
    q = jnp.maximum(m, 0.2 * m)
    t_o[...] = t
    as_o[...] = asv
    ad_o[...] = adv
    adq_o[...] = _leaky20(adv) + q
    q_o[...] = jnp.full((16,), 1.0, _f32) * q


def _k6_body(acc_a, acc_b, den_a, den_b, t, asv, adv, adqv, b2, w3, dis,
             g3_o):
    exs = jnp.exp(_leaky20(asv[...] + adv[...]) - adqv[...])
    den = jnp.maximum(den_a[...] + den_b[...] + exs, 1e-16)
    num = acc_a[...] + acc_b[...] + exs * t[...]
    h2 = _leaky01(num / den + b2[...][None, :])
    p3 = jnp.dot(h2, w3[...], preferred_element_type=_f32)
    g3_o[...] = dis[...] * p3


def _k8_body(acc, g3, dis, b3, batch, wl, bl, out):
    h3 = _leaky01(dis[...] * (acc[0:N, :] + acc[NP:NP + N, :] + g3[...])
                  + b3[...][None, :])
    oh = (batch[...] == lax.broadcasted_iota(jnp.int32, (N, G), 1))
    oh = oh.astype(_f32)
    sums = lax.dot_general(oh, h3, (((0,), (0,)), ((), ())),
                           preferred_element_type=_f32)
    cnt = jnp.sum(oh, axis=0)[:, None]
    pooled = sums / jnp.maximum(cnt, 1.0)
    out[...] = jnp.dot(pooled, wl[...], preferred_element_type=_f32) \
        + bl[...][None, :]


def _tc(body, out_shape, *args):
    return pl.pallas_call(body, out_shape=out_shape)(*args)


def kernel(x, edge_index, batch, W1, b1, W2, a_src, a_dst, b2, W3, b3, Wl,
           bl):
    e = edge_index.shape[1]
    cpt = -(-e // (CH * NW))
    cpt = -(-cpt // 8) * 8
    epad = cpt * NW * CH
    pad = epad - e
    ar = jnp.arange(pad, dtype=jnp.int32)
    srcp = jnp.concatenate([edge_index[0], ar % N]).reshape(epad // CH, CH)
    dstp = jnp.concatenate([edge_index[1], N + (ar % 32)]).reshape(
        epad // CH, CH)

    degp = _deg_call(dstp, cpt)
    dega = degp[:N][:, None]
    degb = degp[NP:NP + N][:, None]

    g1, dis = _tc(_k2_body,
                  (jax.ShapeDtypeStruct((N, 128), _f32),
                   jax.ShapeDtypeStruct((N, 1), _f32)),
                  x, W1, dega, degb)

    acc1 = _gcn_call(g1, srcp, dstp, cpt, 128)
    t, asv, adv, adqv, q16 = _tc(
        _k4_body,
        (jax.ShapeDtypeStruct((N, 64), _f32),
         jax.ShapeDtypeStruct((N, 1), _f32),
         jax.ShapeDtypeStruct((N, 1), _f32),
         jax.ShapeDtypeStruct((N, 1), _f32),
         jax.ShapeDtypeStruct((16,), _f32)),
        acc1, g1, dis, b1, W2, a_src, a_dst)

    zpad = jnp.zeros((NP - N,), _f32)
    asp = asv[:, 0]
    adp = jnp.concatenate([adv[:, 0], zpad])

    denp, acc2 = _gat_call(asp, adp, q16, t, srcp, dstp, cpt, 64)

    g3 = _tc(_k6_body, jax.ShapeDtypeStruct((N, 64), _f32),
             acc2[:N], acc2[NP:NP + N],
             denp[:N][:, None], denp[NP:NP + N][:, None],
             t, asv, adv, adqv, b2, W3, dis)

    acc3 = _gcn_call(g3, srcp, dstp, cpt, 64)

    out = _tc(_k8_body, jax.ShapeDtypeStruct((G, 1), _f32),
              acc3, g3, dis, b3, batch[:, None], Wl, bl)
    return out


# speedup vs baseline: 1.5876x; 1.0011x over previous
import functools

import jax
import jax.numpy as jnp
from jax import lax
from jax.experimental import pallas as pl
from jax.experimental.pallas import tpu as pltpu
from jax.experimental.pallas import tpu_sc as plsc

N = 10000
G = 16
NC = 2
NS = 16
NW = NC * NS
CH = 128
RPT = 640
NP = NS * RPT

_f32 = jnp.float32
_mesh = plsc.VectorSubcoreMesh(core_axis_name="c", subcore_axis_name="s")
_sc_params = pltpu.CompilerParams(use_tc_tiling_on_sc=False)


def _fill_1d(ref, n, val):
    def body(i, _):
        ref[pl.ds(i * 16, 16)] = jnp.full((16,), val, _f32)
        return _
    lax.fori_loop(0, n // 16, body, 0)


def _fill_rows(ref, nrows, d, val):
    def body(r, _):
        for c in range(d // 16):
            ref[r, pl.ds(c * 16, 16)] = jnp.full((16,), val, _f32)
        return _
    lax.fori_loop(0, nrows, body, 0)


def _wid(cid, sid):
    return sid * NC + cid


def _deg_body(cpt, dstp, out, deg_sh, dstv, ones_v, zer_v, sem):
    cid = lax.axis_index("c")
    sid = lax.axis_index("s")
    w = _wid(cid, sid)
    _fill_1d(zer_v, RPT, 0.0)
    _fill_1d(ones_v, CH, 1.0)
    pltpu.sync_copy(zer_v, deg_sh.at[pl.ds(sid * RPT, RPT)])
    plsc.subcore_barrier()
    pltpu.sync_copy(dstp.at[pl.ds(w * cpt, cpt)], dstv)

    def step(j, _):
        pltpu.sync_copy(ones_v, deg_sh.at[dstv.at[j]], add=True)
        return _
    lax.fori_loop(0, cpt, step, 0)
    plsc.subcore_barrier()
    pltpu.sync_copy(deg_sh.at[pl.ds(sid * RPT, RPT)],
                    out.at[pl.ds(cid * NP + sid * RPT, RPT)])


def _deg_call(dstp, cpt):
    return pl.kernel(
        functools.partial(_deg_body, cpt),
        out_type=jax.ShapeDtypeStruct((NC * NP,), _f32),
        mesh=_mesh,
        compiler_params=_sc_params,
        scratch_types=[
            pltpu.VMEM_SHARED((NP,), _f32),
            pltpu.VMEM((cpt, CH), jnp.int32),
            pltpu.VMEM((CH,), _f32),
            pltpu.VMEM((RPT,), _f32),
            pltpu.SemaphoreType.DMA,
        ],
    )(dstp)


_SS = 16


def _gcn_body(cpt, d, g, srcp, dstp, out, acc_sh, srcw, dstw, rows0, rows1,
              semi, sem0, sem1):
    cid = lax.axis_index("c")
    sid = lax.axis_index("s")
    w = _wid(cid, sid)
    nss = cpt // _SS
    _fill_rows(rows0, CH, d, 0.0)
    for r in range(RPT // CH):
        pltpu.sync_copy(rows0, acc_sh.at[pl.ds(sid * RPT + r * CH, CH)])
    plsc.subcore_barrier()
    pltpu.sync_copy(srcp.at[pl.ds(w * cpt, _SS)], srcw.at[0])
    pltpu.sync_copy(dstp.at[pl.ds(w * cpt, _SS)], dstw.at[0])

    def ss_body(p, _):
        b = p % 2
        nb = 1 - b
        off = w * cpt + jnp.minimum((p + 1) * _SS, cpt - _SS)
        pltpu.async_copy(srcp.at[pl.ds(off, _SS)], srcw.at[nb], semi)
        pltpu.async_copy(dstp.at[pl.ds(off, _SS)], dstw.at[nb], semi)
        bufs = (rows0, rows1)
        sems = (sem0, sem1)
        pltpu.async_copy(g.at[srcw.at[b, 0]], rows0, sem0)
        pltpu.async_copy(g.at[srcw.at[b, 1]], rows1, sem1)
        for k in range(_SS):
            rb, smb = bufs[k % 2], sems[k % 2]
            pltpu.make_async_copy(g.at[srcw.at[b, k]], rb, smb).wait()
            pltpu.sync_copy(rb, acc_sh.at[dstw.at[b, k]], add=True)
            if k + 2 < _SS:
                pltpu.async_copy(g.at[srcw.at[b, k + 2]], rb, smb)
        pltpu.make_async_copy(srcp.at[pl.ds(off, _SS)], srcw.at[nb],
                              semi).wait()
        pltpu.make_async_copy(dstp.at[pl.ds(off, _SS)], dstw.at[nb],
                              semi).wait()
        return _
    lax.fori_loop(0, nss, ss_body, 0)
    plsc.subcore_barrier()
    for r in range(RPT // CH):
        pltpu.sync_copy(acc_sh.at[pl.ds(sid * RPT + r * CH, CH)],
                        out.at[pl.ds(cid * NP + sid * RPT + r * CH, CH)])


def _gcn_call(g, srcp, dstp, cpt, d):
    return pl.kernel(
        functools.partial(_gcn_body, cpt, d),
        out_type=jax.ShapeDtypeStruct((NC * NP, d), _f32),
        mesh=_mesh,
        compiler_params=_sc_params,
        scratch_types=[
            pltpu.VMEM_SHARED((NP, d), _f32),
            pltpu.VMEM((2, _SS, CH), jnp.int32),
            pltpu.VMEM((2, _SS, CH), jnp.int32),
            pltpu.VMEM((CH, d), _f32),
            pltpu.VMEM((CH, d), _f32),
            pltpu.SemaphoreType.DMA,
            pltpu.SemaphoreType.DMA,
            pltpu.SemaphoreType.DMA,
        ],
    )(g, srcp, dstp)


def _gat_body(cpt, d, asp, adp, qh, t, srcp, dstp, den_out, acc_out,
              den_sh, acc_sh, srcv, dstv, asv0, asv1, adv0, adv1,
              exv0, exv1, rows0, rows1, qv, zer_v, sem0, sem1, semd):
    cid = lax.axis_index("c")
    sid = lax.axis_index("s")
    w = _wid(cid, sid)
    _fill_1d(zer_v, RPT, 0.0)
    pltpu.sync_copy(zer_v, den_sh.at[pl.ds(sid * RPT, RPT)])
    _fill_rows(rows0, CH, d, 0.0)
    for r in range(RPT // CH):
        pltpu.sync_copy(rows0, acc_sh.at[pl.ds(sid * RPT + r * CH, CH)])
    plsc.subcore_barrier()
    pltpu.sync_copy(srcp.at[pl.ds(w * cpt, cpt)], srcv.at[pl.ds(0, cpt)])
    for rr in (cpt, cpt + 1):
        for c in range(CH // 16):
            srcv[rr, pl.ds(c * 16, 16)] = jnp.zeros((16,), jnp.int32)
            dstv[rr, pl.ds(c * 16, 16)] = jnp.zeros((16,), jnp.int32)
    pltpu.sync_copy(dstp.at[pl.ds(w * cpt, cpt)], dstv.at[pl.ds(0, cpt)])
    pltpu.sync_copy(qh, qv)
    qq = qv[...]

    def start(j, asv, adv, rows, sem):
        pltpu.async_copy(asp.at[srcv.at[j]], asv, sem)
        pltpu.async_copy(adp.at[dstv.at[j]], adv, sem)
        pltpu.async_copy(t.at[srcv.at[j]], rows, sem)

    def wait(j, asv, adv, rows, sem):
        pltpu.make_async_copy(asp.at[srcv.at[j]], asv, sem).wait()
        pltpu.make_async_copy(adp.at[dstv.at[j]], adv, sem).wait()
        pltpu.make_async_copy(t.at[srcv.at[j]], rows, sem).wait()

    def compute(j, asv, adv, exv, rows):
        pltpu.make_async_copy(exv, den_sh.at[dstv.at[j]], semd).wait()
        for gidx in range(CH // 16):
            sl = pl.ds(gidx * 16, 16)
            u = asv[sl] + adv[sl]
            e = jnp.where(u > 0, u, 0.2 * u)
            ad = adv[sl]
            c_sh = jnp.where(ad > 0, ad, 0.2 * ad) + qq
            exv[sl] = jnp.exp(e - c_sh)
        pltpu.async_copy(exv, den_sh.at[dstv.at[j]], semd, add=True)

        def scale(gi, _c):
            exg = exv[pl.ds(gi * 16, 16)]
            for l in range(16):
                k = gi * 16 + l
                exb = jnp.full((16,), 1.0, _f32) * exg[l]
                for c in range(d // 16):
                    csl = pl.ds(c * 16, 16)
                    rows[k, csl] = rows[k, csl] * exb
            return _c
        lax.fori_loop(0, CH // 16, scale, 0)
        pltpu.sync_copy(rows, acc_sh.at[dstv.at[j]], add=True)

    _fill_1d(exv0, CH, 0.0)
    _fill_1d(exv1, CH, 0.0)
    pltpu.async_copy(exv0, den_sh.at[dstv.at[cpt]], semd, add=True)
    pltpu.async_copy(exv1, den_sh.at[dstv.at[cpt]], semd, add=True)
    start(0, asv0, adv0, rows0, sem0)

    def step(p, _):
        j0 = p * 2
        start(j0 + 1, asv1, adv1, rows1, sem1)
        wait(j0, asv0, adv0, rows0, sem0)
        compute(j0, asv0, adv0, exv0, rows0)
        start(j0 + 2, asv0, adv0, rows0, sem0)
        wait(j0 + 1, asv1, adv1, rows1, sem1)
        compute(j0 + 1, asv1, adv1, exv1, rows1)
        return _
    lax.fori_loop(0, cpt // 2, step, 0)
    wait(cpt, asv0, adv0, rows0, sem0)
    pltpu.make_async_copy(exv0, den_sh.at[dstv.at[cpt]], semd).wait()
    pltpu.make_async_copy(exv1, den_sh.at[dstv.at[cpt]], semd).wait()
    plsc.subcore_barrier()
    pltpu.sync_copy(den_sh.at[pl.ds(sid * RPT, RPT)],
                    den_out.at[pl.ds(cid * NP + sid * RPT, RPT)])
    for r in range(RPT // CH):
        pltpu.sync_copy(acc_sh.at[pl.ds(sid * RPT + r * CH, CH)],
                        acc_out.at[pl.ds(cid * NP + sid * RPT + r * CH, CH)])


def _gat_call(asp, adp, qh, t, srcp, dstp, cpt, d):
    return pl.kernel(
        functools.partial(_gat_body, cpt, d),
        out_type=(jax.ShapeDtypeStruct((NC * NP,), _f32),
                  jax.ShapeDtypeStruct((NC * NP, d), _f32)),
        mesh=_mesh,
        compiler_params=_sc_params,
        scratch_types=[
            pltpu.VMEM_SHARED((NP,), _f32),
            pltpu.VMEM_SHARED((NP, d), _f32),
            pltpu.VMEM((cpt + 2, CH), jnp.int32),
            pltpu.VMEM((cpt + 2, CH), jnp.int32),
            pltpu.VMEM((CH,), _f32),
            pltpu.VMEM((CH,), _f32),
            pltpu.VMEM((CH,), _f32),
            pltpu.VMEM((CH,), _f32),
            pltpu.VMEM((CH,), _f32),
            pltpu.VMEM((CH,), _f32),
            pltpu.VMEM((CH, d), _f32),
            pltpu.VMEM((CH, d), _f32),
            pltpu.VMEM((16,), _f32),
            pltpu.VMEM((RPT,), _f32),
            pltpu.SemaphoreType.DMA,
            pltpu.SemaphoreType.DMA,
            pltpu.SemaphoreType.DMA,
        ],
    )(asp, adp, qh, t, srcp, dstp)


def _leaky01(v):
    return jnp.where(v > 0, v, 0.01 * v)


def _leaky20(v):
    return jnp.where(v > 0, v, 0.2 * v)


def _k2_body(x_ref, w1_ref, dega_ref, degb_ref, g1_ref, dis_ref):
    deg = dega_ref[...] + degb_ref[...] + 1.0
    dis = lax.rsqrt(deg)
    p1 = jnp.dot(x_ref[...], w1_ref[...], preferred_element_type=_f32)
    g1_ref[...] = dis * p1
    dis_ref[...] = dis


def _k4_body(acc, g1, dis, b1, w2, a_s, a_d, t_o, as_o, ad_o,
             adq_o, q_o):
    h1 = _leaky01(dis[...] * (acc[0:N, :] + acc[NP:NP + N, :] + g1[...])
                  + b1[...][None, :])
    t = jnp.dot(h1, w2[...], preferred_element_type=_f32)
    asv = jnp.dot(t, a_s[...][:, None], preferred_element_type=_f32)
    adv = jnp.dot(t, a_d[...][:, None], preferred_element_type=_f32)
    m = jnp.max(asv)
    q = jnp.maximum(m, 0.2 * m)
    t_o[...] = t
    as_o[...] = asv
    ad_o[...] = adv
    adq_o[...] = _leaky20(adv) + q
    q_o[...] = jnp.full((16,), 1.0, _f32) * q


def _k6_body(acc_a, acc_b, den_a, den_b, t, asv, adv, adqv, b2, w3, dis,
             g3_o):
    exs = jnp.exp(_leaky20(asv[...] + adv[...]) - adqv[...])
    den = jnp.maximum(den_a[...] + den_b[...] + exs, 1e-16)
    num = acc_a[...] + acc_b[...] + exs * t[...]
    h2 = _leaky01(num / den + b2[...][None, :])
    p3 = jnp.dot(h2, w3[...], preferred_element_type=_f32)
    g3_o[...] = dis[...] * p3


def _k8_body(acc, g3, dis, b3, batch, wl, bl, out):
    h3 = _leaky01(dis[...] * (acc[0:N, :] + acc[NP:NP + N, :] + g3[...])
                  + b3[...][None, :])
    oh = (batch[...] == lax.broadcasted_iota(jnp.int32, (N, G), 1))
    oh = oh.astype(_f32)
    sums = lax.dot_general(oh, h3, (((0,), (0,)), ((), ())),
                           preferred_element_type=_f32)
    cnt = jnp.sum(oh, axis=0)[:, None]
    pooled = sums / jnp.maximum(cnt, 1.0)
    out[...] = jnp.dot(pooled, wl[...], preferred_element_type=_f32) \
        + bl[...][None, :]


def _tc(body, out_shape, *args):
    return pl.pallas_call(body, out_shape=out_shape)(*args)


def kernel(x, edge_index, batch, W1, b1, W2, a_src, a_dst, b2, W3, b3, Wl,
           bl):
    e = edge_index.shape[1]
    cpt = -(-e // (CH * NW))
    cpt = -(-cpt // 8) * 8
    epad = cpt * NW * CH
    pad = epad - e
    ar = jnp.arange(pad, dtype=jnp.int32)
    srcp = jnp.concatenate([edge_index[0], ar % N]).reshape(epad // CH, CH)
    dstp = jnp.concatenate([edge_index[1], N + (ar % 32)]).reshape(
        epad // CH, CH)

    degp = _deg_call(dstp, cpt)
    dega = degp[:N][:, None]
    degb = degp[NP:NP + N][:, None]

    g1, dis = _tc(_k2_body,
                  (jax.ShapeDtypeStruct((N, 128), _f32),
                   jax.ShapeDtypeStruct((N, 1), _f32)),
                  x, W1, dega, degb)

    acc1 = _gcn_call(g1, srcp, dstp, cpt, 128)
    t, asv, adv, adqv, q16 = _tc(
        _k4_body,
        (jax.ShapeDtypeStruct((N, 64), _f32),
         jax.ShapeDtypeStruct((N, 1), _f32),
         jax.ShapeDtypeStruct((N, 1), _f32),
         jax.ShapeDtypeStruct((N, 1), _f32),
         jax.ShapeDtypeStruct((16,), _f32)),
        acc1, g1, dis, b1, W2, a_src, a_dst)

    zpad = jnp.zeros((NP - N,), _f32)
    asp = asv[:, 0]
    adp = jnp.concatenate([adv[:, 0], zpad])

    denp, acc2 = _gat_call(asp, adp, q16, t, srcp, dstp, cpt, 64)

    g3 = _tc(_k6_body, jax.ShapeDtypeStruct((N, 64), _f32),
             acc2[:N], acc2[NP:NP + N],
             denp[:N][:, None], denp[NP:NP + N][:, None],
             t, asv, adv, adqv, b2, W3, dis)

    acc3 = _gcn_call(g3, srcp, dstp, cpt, 64)

    out = _tc(_k8_body, jax.ShapeDtypeStruct((G, 1), _f32),
              acc3, g3, dis, b3, batch[:, None], Wl, bl)
    return out
